# baseline (speedup 1.0000x reference)
import jax
import jax.numpy as jnp
from jax import lax
from jax.experimental import pallas as pl
from jax.experimental.pallas import tpu as pltpu
from jax.experimental.pallas import tpu_sc as plsc

N = 10000
E = 320000
D = 128

NC = 2
NS = 16
NW = NC * NS
EPW = E // NW
CHUNK = 125
NCHUNKS = EPW // CHUNK
WB = 80
NWB = N // WB
NH = 2
HC = NCHUNKS // NH
NQ = 5
QC = NCHUNKS // NQ


def _fill(buf, value):
    @pl.loop(0, CHUNK)
    def _(i):
        for j in range(D // 16):
            buf[i, pl.ds(j * 16, 16)] = jnp.full((16,), value, jnp.float32)


def _agg_body(x_hbm, eidx_hbm, psum_hbm, pcnt_hbm,
              src_i, dst_i, dst_q, rows_a, rows_b, acc_sh, sg_a, sg_b):
    c = lax.axis_index("c")
    s = lax.axis_index("s")
    wid = c * NS + s

    _fill(rows_b, 0.0)

    @pl.loop(s, NWB, step=NS)
    def _(t):
        pltpu.sync_copy(rows_b.at[pl.ds(0, WB)],
                        acc_sh.at[pl.ds(t * WB, WB)])

    plsc.subcore_barrier()

    pltpu.sync_copy(eidx_hbm.at[1, wid], dst_i)

    def gather(j, buf, sem):
        pltpu.async_copy(x_hbm.at[src_i.at[j]], buf, sem)

    def gwait(j, buf, sem):
        pltpu.make_async_copy(x_hbm.at[src_i.at[j]], buf, sem).wait()

    bufs = [(rows_a, sg_a), (rows_b, sg_b)]
    NB = len(bufs)

    for h in range(NH):
        d0 = h * HC
        pltpu.sync_copy(eidx_hbm.at[0, wid, pl.ds(d0, HC)], src_i)
        for k in range(NB):
            gather(k, *bufs[k])

        @pl.loop(0, HC // NB - 1)
        def _(jg):
            j = jg * NB
            for k in range(NB):
                buf, sem = bufs[k]
                gwait(j + k, buf, sem)
                pltpu.sync_copy(buf, acc_sh.at[dst_i.at[d0 + j + k]],
                                add=True)
                gather(j + NB + k, buf, sem)

        jl = HC - NB
        for k in range(NB):
            buf, sem = bufs[k]
            gwait(jl + k, buf, sem)
            pltpu.sync_copy(buf, acc_sh.at[dst_i.at[d0 + jl + k]],
                            add=True)

    plsc.subcore_barrier()

    _fill(rows_b, 0.0)

    @pl.loop(s, NWB, step=NS)
    def _(t):
        sl = pl.ds(t * WB, WB)
        pltpu.sync_copy(acc_sh.at[sl], rows_a.at[pl.ds(0, WB)])
        pltpu.sync_copy(rows_a.at[pl.ds(0, WB)], psum_hbm.at[c, sl])
        pltpu.sync_copy(rows_b.at[pl.ds(0, WB)], acc_sh.at[sl])

    plsc.subcore_barrier()

    _fill(rows_a, 1.0)

    def cscat(j):
        pltpu.async_copy(rows_a, acc_sh.at[dst_q.at[j]], sg_a, add=True)

    def cwait(j):
        pltpu.make_async_copy(rows_a, acc_sh.at[dst_q.at[j]], sg_a).wait()

    DEPTH = 3
    for q in range(NQ):
        pltpu.sync_copy(eidx_hbm.at[1, wid, pl.ds(q * QC, QC)], dst_q)
        for k in range(DEPTH):
            cscat(k)

        @pl.loop(0, QC - DEPTH)
        def _(j):
            cscat(j + DEPTH)
            cwait(j)

        for k in range(QC - DEPTH, QC):
            cwait(k)

    plsc.subcore_barrier()

    @pl.loop(s, NWB, step=NS)
    def _(t):
        sl = pl.ds(t * WB, WB)
        pltpu.sync_copy(acc_sh.at[sl], rows_b.at[pl.ds(0, WB)])
        pltpu.sync_copy(rows_b.at[pl.ds(0, WB)], pcnt_hbm.at[c, sl])


_sc_agg = pl.kernel(
    _agg_body,
    out_type=[
        jax.ShapeDtypeStruct((NC, N, D), jnp.float32),
        jax.ShapeDtypeStruct((NC, N, D), jnp.float32),
    ],
    mesh=plsc.VectorSubcoreMesh(core_axis_name="c", subcore_axis_name="s"),
    scratch_types=[
        pltpu.VMEM((HC, CHUNK), jnp.int32),
        pltpu.VMEM((NCHUNKS, CHUNK), jnp.int32),
        pltpu.VMEM((QC, CHUNK), jnp.int32),
        pltpu.VMEM((CHUNK, D), jnp.float32),
        pltpu.VMEM((CHUNK, D), jnp.float32),
        pltpu.VMEM_SHARED((N, D), jnp.float32),
        pltpu.SemaphoreType.DMA,
        pltpu.SemaphoreType.DMA,
    ],
)


BN = 1000


def _tc_body(psum_ref, pcnt_ref, x_ref, wl_ref, bl_ref, wr_ref, out_ref):
    ssum = psum_ref[0] + psum_ref[1]
    cnt = pcnt_ref[0][:, 0:1] + pcnt_ref[1][:, 0:1]
    mean = ssum / jnp.maximum(cnt, 1.0)
    out = (jnp.dot(mean, wl_ref[:], preferred_element_type=jnp.float32)
           + bl_ref[:]
           + jnp.dot(x_ref[:], wr_ref[:], preferred_element_type=jnp.float32))
    nrm = jnp.sqrt(jnp.sum(out * out, axis=1, keepdims=True))
    out_ref[:] = out / jnp.maximum(nrm, 1e-12)


def _tc_dense(psum, pcnt, x, W_l, b_l, W_r):
    return pl.pallas_call(
        _tc_body,
        grid=(N // BN,),
        in_specs=[
            pl.BlockSpec((NC, BN, D), lambda i: (0, i, 0)),
            pl.BlockSpec((NC, BN, D), lambda i: (0, i, 0)),
            pl.BlockSpec((BN, D), lambda i: (i, 0)),
            pl.BlockSpec((D, D), lambda i: (0, 0)),
            pl.BlockSpec((1, D), lambda i: (0, 0)),
            pl.BlockSpec((D, D), lambda i: (0, 0)),
        ],
        out_specs=pl.BlockSpec((BN, D), lambda i: (i, 0)),
        out_shape=jax.ShapeDtypeStruct((N, D), jnp.float32),
    )(psum, pcnt, x, W_l, b_l, W_r)


def kernel(x, edge_index, W_l, b_l, W_r):
    eidx = edge_index.reshape(2, NW, NCHUNKS, CHUNK)
    psum, pcnt = _sc_agg(x, eidx)
    return _tc_dense(psum, pcnt, x, W_l, b_l.reshape(1, D), W_r)


# speedup vs baseline: 1.0084x; 1.0084x over previous
import jax
import jax.numpy as jnp
from jax import lax
from jax.experimental import pallas as pl
from jax.experimental.pallas import tpu as pltpu
from jax.experimental.pallas import tpu_sc as plsc

N = 10000
E = 320000
D = 128

NC = 2
NS = 16
NW = NC * NS
EPW = E // NW
CHUNK = 125
NCHUNKS = EPW // CHUNK
WB = 80
NWB = N // WB
NH = 2
HC = NCHUNKS // NH


def _fill(buf, value):
    @pl.loop(0, CHUNK)
    def _(i):
        for j in range(D // 16):
            buf[i, pl.ds(j * 16, 16)] = jnp.full((16,), value, jnp.float32)


def _agg_body(x_hbm, eidx_hbm, psum_hbm, pcnt_hbm,
              src_i, dst_i, rows_a, rows_b, acc_sh, sg_a, sg_b):
    c = lax.axis_index("c")
    s = lax.axis_index("s")
    wid = c * NS + s

    _fill(rows_b, 0.0)

    @pl.loop(s, NWB, step=NS)
    def _(t):
        pltpu.sync_copy(rows_b.at[pl.ds(0, WB)],
                        acc_sh.at[pl.ds(t * WB, WB)])

    plsc.subcore_barrier()

    pltpu.sync_copy(eidx_hbm.at[1, wid], dst_i)

    def gather(j, buf, sem):
        pltpu.async_copy(x_hbm.at[src_i.at[j]], buf, sem)

    def gwait(j, buf, sem):
        pltpu.make_async_copy(x_hbm.at[src_i.at[j]], buf, sem).wait()

    bufs = [(rows_a, sg_a), (rows_b, sg_b)]
    NB = len(bufs)

    for h in range(NH):
        d0 = h * HC
        pltpu.sync_copy(eidx_hbm.at[0, wid, pl.ds(d0, HC)], src_i)
        for k in range(NB):
            gather(k, *bufs[k])

        @pl.loop(0, HC // NB - 1)
        def _(jg):
            j = jg * NB
            for k in range(NB):
                buf, sem = bufs[k]
                gwait(j + k, buf, sem)
                pltpu.sync_copy(buf, acc_sh.at[dst_i.at[d0 + j + k]],
                                add=True)
                gather(j + NB + k, buf, sem)

        jl = HC - NB
        for k in range(NB):
            buf, sem = bufs[k]
            gwait(jl + k, buf, sem)
            pltpu.sync_copy(buf, acc_sh.at[dst_i.at[d0 + jl + k]],
                            add=True)

    plsc.subcore_barrier()

    _fill(rows_b, 0.0)

    @pl.loop(s, NWB, step=NS)
    def _(t):
        sl = pl.ds(t * WB, WB)
        pltpu.sync_copy(acc_sh.at[sl], rows_a.at[pl.ds(0, WB)])
        pltpu.sync_copy(rows_a.at[pl.ds(0, WB)], psum_hbm.at[c, sl])
        pltpu.sync_copy(rows_b.at[pl.ds(0, WB)], acc_sh.at[sl])

    plsc.subcore_barrier()

    _fill(rows_a, 1.0)

    @pl.loop(0, NCHUNKS)
    def _(j):
        pltpu.sync_copy(rows_a, acc_sh.at[dst_i.at[j]], add=True)

    plsc.subcore_barrier()

    @pl.loop(s, NWB, step=NS)
    def _(t):
        sl = pl.ds(t * WB, WB)
        pltpu.sync_copy(acc_sh.at[sl], rows_b.at[pl.ds(0, WB)])
        pltpu.sync_copy(rows_b.at[pl.ds(0, WB)], pcnt_hbm.at[c, sl])


_sc_agg = pl.kernel(
    _agg_body,
    out_type=[
        jax.ShapeDtypeStruct((NC, N, D), jnp.float32),
        jax.ShapeDtypeStruct((NC, N, D), jnp.float32),
    ],
    mesh=plsc.VectorSubcoreMesh(core_axis_name="c", subcore_axis_name="s"),
    scratch_types=[
        pltpu.VMEM((HC, CHUNK), jnp.int32),
        pltpu.VMEM((NCHUNKS, CHUNK), jnp.int32),
        pltpu.VMEM((CHUNK, D), jnp.float32),
        pltpu.VMEM((CHUNK, D), jnp.float32),
        pltpu.VMEM_SHARED((N, D), jnp.float32),
        pltpu.SemaphoreType.DMA,
        pltpu.SemaphoreType.DMA,
    ],
)


BN = 1000


def _tc_xr_body(x_ref, wr_ref, bl_ref, out_ref):
    out_ref[:] = (jnp.dot(x_ref[:], wr_ref[:],
                          preferred_element_type=jnp.float32) + bl_ref[:])


def _tc_xr(x, W_r, b_l):
    return pl.pallas_call(
        _tc_xr_body,
        grid=(N // BN,),
        in_specs=[
            pl.BlockSpec((BN, D), lambda i: (i, 0)),
            pl.BlockSpec((D, D), lambda i: (0, 0)),
            pl.BlockSpec((1, D), lambda i: (0, 0)),
        ],
        out_specs=pl.BlockSpec((BN, D), lambda i: (i, 0)),
        out_shape=jax.ShapeDtypeStruct((N, D), jnp.float32),
    )(x, W_r, b_l)


def _tc_body(psum_ref, pcnt_ref, xr_ref, wl_ref, out_ref):
    ssum = psum_ref[0] + psum_ref[1]
    cnt = pcnt_ref[0][:, 0:1] + pcnt_ref[1][:, 0:1]
    mean = ssum / jnp.maximum(cnt, 1.0)
    out = (jnp.dot(mean, wl_ref[:], preferred_element_type=jnp.float32)
           + xr_ref[:])
    nrm = jnp.sqrt(jnp.sum(out * out, axis=1, keepdims=True))
    out_ref[:] = out / jnp.maximum(nrm, 1e-12)


def _tc_dense(psum, pcnt, xr, W_l):
    return pl.pallas_call(
        _tc_body,
        grid=(N // BN,),
        in_specs=[
            pl.BlockSpec((NC, BN, D), lambda i: (0, i, 0)),
            pl.BlockSpec((NC, BN, D), lambda i: (0, i, 0)),
            pl.BlockSpec((BN, D), lambda i: (i, 0)),
            pl.BlockSpec((D, D), lambda i: (0, 0)),
        ],
        out_specs=pl.BlockSpec((BN, D), lambda i: (i, 0)),
        out_shape=jax.ShapeDtypeStruct((N, D), jnp.float32),
    )(psum, pcnt, xr, W_l)


def kernel(x, edge_index, W_l, b_l, W_r):
    eidx = edge_index.reshape(2, NW, NCHUNKS, CHUNK)
    psum, pcnt = _sc_agg(x, eidx)
    xr = _tc_xr(x, W_r, b_l.reshape(1, D))
    return _tc_dense(psum, pcnt, xr, W_l)


# speedup vs baseline: 1.0300x; 1.0214x over previous
import jax
import jax.numpy as jnp
from jax import lax
from jax.experimental import pallas as pl
from jax.experimental.pallas import tpu as pltpu
from jax.experimental.pallas import tpu_sc as plsc

N = 10000
E = 320000
D = 128

NC = 2
NS = 16
NW = NC * NS
EPW = E // NW
CHUNK = 125
NCHUNKS = EPW // CHUNK
WB = 80
NWB = N // WB
NH = 2
HC = NCHUNKS // NH


def _fill(buf, value):
    @pl.loop(0, CHUNK)
    def _(i):
        for j in range(D // 16):
            buf[i, pl.ds(j * 16, 16)] = jnp.full((16,), value, jnp.float32)


def _agg_body(x_hbm, eidx_hbm, psum_hbm, pcnt_hbm,
              src_i, dst_i, rows_a, rows_b, acc_sh, sg_a, sg_b):
    c = lax.axis_index("c")
    s = lax.axis_index("s")
    wid = c * NS + s

    _fill(rows_b, 0.0)

    @pl.loop(s, NWB, step=NS)
    def _(t):
        pltpu.sync_copy(rows_b.at[pl.ds(0, WB)],
                        acc_sh.at[pl.ds(t * WB, WB)])

    plsc.subcore_barrier()

    pltpu.sync_copy(eidx_hbm.at[1, wid], dst_i)

    def gather(j, buf, sem):
        pltpu.async_copy(x_hbm.at[src_i.at[j]], buf, sem)

    def gwait(j, buf, sem):
        pltpu.make_async_copy(x_hbm.at[src_i.at[j]], buf, sem).wait()

    bufs = [(rows_a, sg_a), (rows_b, sg_b)]
    NB = len(bufs)

    for h in range(NH):
        d0 = h * HC
        pltpu.sync_copy(eidx_hbm.at[0, wid, pl.ds(d0, HC)], src_i)
        for k in range(NB):
            gather(k, *bufs[k])

        @pl.loop(0, HC // NB - 1)
        def _(jg):
            j = jg * NB
            for k in range(NB):
                buf, sem = bufs[k]
                gwait(j + k, buf, sem)
                pltpu.sync_copy(buf, acc_sh.at[dst_i.at[d0 + j + k]],
                                add=True)
                gather(j + NB + k, buf, sem)

        jl = HC - NB
        for k in range(NB):
            buf, sem = bufs[k]
            gwait(jl + k, buf, sem)
            pltpu.sync_copy(buf, acc_sh.at[dst_i.at[d0 + jl + k]],
                            add=True)

    plsc.subcore_barrier()

    @pl.loop(s, NWB, step=NS)
    def _(t):
        sl = pl.ds(t * WB, WB)
        pltpu.sync_copy(acc_sh.at[sl], rows_a.at[pl.ds(0, WB)])
        pltpu.sync_copy(rows_a.at[pl.ds(0, WB)], psum_hbm.at[c, sl])

    plsc.subcore_barrier()

    _fill(rows_a, 1.0)

    @pl.loop(0, NCHUNKS)
    def _(j):
        pltpu.sync_copy(rows_a, acc_sh.at[dst_i.at[j]], add=True)

    plsc.subcore_barrier()

    @pl.loop(s, NWB, step=NS)
    def _(t):
        sl = pl.ds(t * WB, WB)
        pltpu.sync_copy(acc_sh.at[sl], rows_b.at[pl.ds(0, WB)])
        pltpu.sync_copy(rows_b.at[pl.ds(0, WB)], pcnt_hbm.at[c, sl])


_sc_agg = pl.kernel(
    _agg_body,
    out_type=[
        jax.ShapeDtypeStruct((NC, N, D), jnp.float32),
        jax.ShapeDtypeStruct((NC, N, D), jnp.float32),
    ],
    mesh=plsc.VectorSubcoreMesh(core_axis_name="c", subcore_axis_name="s"),
    scratch_types=[
        pltpu.VMEM((HC, CHUNK), jnp.int32),
        pltpu.VMEM((NCHUNKS, CHUNK), jnp.int32),
        pltpu.VMEM((CHUNK, D), jnp.float32),
        pltpu.VMEM((CHUNK, D), jnp.float32),
        pltpu.VMEM_SHARED((N, D), jnp.float32),
        pltpu.SemaphoreType.DMA,
        pltpu.SemaphoreType.DMA,
    ],
)


BN = 1000


def _tc_body(psum_ref, pcnt_ref, x_ref, wl_ref, bl_ref, wr_ref, out_ref):
    ssum = psum_ref[0] + psum_ref[1]
    cnt = (pcnt_ref[0][:, 0:1] + pcnt_ref[1][:, 0:1]) - ssum[:, 0:1]
    mean = ssum / jnp.maximum(cnt, 1.0)
    out = (jnp.dot(mean, wl_ref[:], preferred_element_type=jnp.float32)
           + bl_ref[:]
           + jnp.dot(x_ref[:], wr_ref[:], preferred_element_type=jnp.float32))
    nrm = jnp.sqrt(jnp.sum(out * out, axis=1, keepdims=True))
    out_ref[:] = out / jnp.maximum(nrm, 1e-12)


def _tc_dense(psum, pcnt, x, W_l, b_l, W_r):
    return pl.pallas_call(
        _tc_body,
        grid=(N // BN,),
        in_specs=[
            pl.BlockSpec((NC, BN, D), lambda i: (0, i, 0)),
            pl.BlockSpec((NC, BN, D), lambda i: (0, i, 0)),
            pl.BlockSpec((BN, D), lambda i: (i, 0)),
            pl.BlockSpec((D, D), lambda i: (0, 0)),
            pl.BlockSpec((1, D), lambda i: (0, 0)),
            pl.BlockSpec((D, D), lambda i: (0, 0)),
        ],
        out_specs=pl.BlockSpec((BN, D), lambda i: (i, 0)),
        out_shape=jax.ShapeDtypeStruct((N, D), jnp.float32),
    )(psum, pcnt, x, W_l, b_l, W_r)


def kernel(x, edge_index, W_l, b_l, W_r):
    eidx = edge_index.reshape(2, NW, NCHUNKS, CHUNK)
    psum, pcnt = _sc_agg(x, eidx)
    return _tc_dense(psum, pcnt, x, W_l, b_l.reshape(1, D), W_r)


# speedup vs baseline: 1.0414x; 1.0110x over previous
import jax
import jax.numpy as jnp
from jax import lax
from jax.experimental import pallas as pl
from jax.experimental.pallas import tpu as pltpu
from jax.experimental.pallas import tpu_sc as plsc

N = 10000
E = 320000
D = 128

NC = 2
NS = 16
NW = NC * NS
EPW = E // NW
CHUNK = 125
NCHUNKS = EPW // CHUNK
WB = 80
NWB = N // WB
NH = 2
HC = NCHUNKS // NH


def _fill(buf, value):
    @pl.loop(0, CHUNK)
    def _(i):
        for j in range(D // 16):
            buf[i, pl.ds(j * 16, 16)] = jnp.full((16,), value, jnp.float32)


def _agg_body(x_hbm, eidx_hbm, psum_hbm, pcnt_hbm,
              src_i, dst_i, rows_a, rows_b, acc_sh, sg_a, sg_b):
    c = lax.axis_index("c")
    s = lax.axis_index("s")
    wid = c * NS + s

    _fill(rows_b, 0.0)

    @pl.loop(s, NWB, step=NS)
    def _(t):
        pltpu.sync_copy(rows_b.at[pl.ds(0, WB)],
                        acc_sh.at[pl.ds(t * WB, WB)])

    plsc.subcore_barrier()

    pltpu.sync_copy(eidx_hbm.at[1, wid], dst_i)

    def gather(j, buf, sem):
        pltpu.async_copy(x_hbm.at[src_i.at[j]], buf, sem)

    def gwait(j, buf, sem):
        pltpu.make_async_copy(x_hbm.at[src_i.at[j]], buf, sem).wait()

    bufs = [(rows_a, sg_a), (rows_b, sg_b)]
    NB = len(bufs)

    for h in range(NH):
        d0 = h * HC
        pltpu.sync_copy(eidx_hbm.at[0, wid, pl.ds(d0, HC)], src_i)
        for k in range(NB):
            gather(k, *bufs[k])

        @pl.loop(0, HC // NB - 1)
        def _(jg):
            j = jg * NB
            for k in range(NB):
                buf, sem = bufs[k]
                gwait(j + k, buf, sem)
                pltpu.sync_copy(buf, acc_sh.at[dst_i.at[d0 + j + k]],
                                add=True)
                gather(j + NB + k, buf, sem)

        jl = HC - NB
        for k in range(NB):
            buf, sem = bufs[k]
            gwait(jl + k, buf, sem)
            pltpu.sync_copy(buf, acc_sh.at[dst_i.at[d0 + jl + k]],
                            add=True)

    plsc.subcore_barrier()

    @pl.loop(s, NWB, step=NS)
    def _(t):
        sl = pl.ds(t * WB, WB)
        pltpu.sync_copy(acc_sh.at[sl], rows_a.at[pl.ds(0, WB)])
        pltpu.sync_copy(rows_a.at[pl.ds(0, WB)], psum_hbm.at[c, sl])

    plsc.subcore_barrier()

    _fill(rows_a, 1.0)

    @pl.loop(0, NCHUNKS)
    def _(j):
        pltpu.sync_copy(rows_a, acc_sh.at[dst_i.at[j]], add=True)

    plsc.subcore_barrier()

    @pl.loop(s, NWB, step=NS)
    def _(t):
        sl = pl.ds(t * WB, WB)
        pltpu.sync_copy(acc_sh.at[sl], rows_b.at[pl.ds(0, WB)])
        pltpu.sync_copy(rows_b.at[pl.ds(0, WB)], pcnt_hbm.at[c, sl])


_sc_agg = pl.kernel(
    _agg_body,
    out_type=[
        jax.ShapeDtypeStruct((NC, N, D), jnp.float32),
        jax.ShapeDtypeStruct((NC, N, D), jnp.float32),
    ],
    mesh=plsc.VectorSubcoreMesh(core_axis_name="c", subcore_axis_name="s"),
    scratch_types=[
        pltpu.VMEM((HC, CHUNK), jnp.int32),
        pltpu.VMEM((NCHUNKS, CHUNK), jnp.int32),
        pltpu.VMEM((CHUNK, D), jnp.float32),
        pltpu.VMEM((CHUNK, D), jnp.float32),
        pltpu.VMEM_SHARED((N, D), jnp.float32),
        pltpu.SemaphoreType.DMA,
        pltpu.SemaphoreType.DMA,
    ],
)


BN = 2000


def _tc_body(psum_ref, pcnt_ref, x_ref, wl_ref, bl_ref, wr_ref, out_ref):
    ssum = psum_ref[0] + psum_ref[1]
    cnt = (pcnt_ref[0][:, 0:1] + pcnt_ref[1][:, 0:1]) - ssum[:, 0:1]
    mean = ssum / jnp.maximum(cnt, 1.0)
    out = (jnp.dot(mean, wl_ref[:], preferred_element_type=jnp.float32)
           + bl_ref[:]
           + jnp.dot(x_ref[:], wr_ref[:], preferred_element_type=jnp.float32))
    nrm = jnp.sqrt(jnp.sum(out * out, axis=1, keepdims=True))
    out_ref[:] = out / jnp.maximum(nrm, 1e-12)


def _tc_dense(psum, pcnt, x, W_l, b_l, W_r):
    return pl.pallas_call(
        _tc_body,
        grid=(N // BN,),
        in_specs=[
            pl.BlockSpec((NC, BN, D), lambda i: (0, i, 0)),
            pl.BlockSpec((NC, BN, D), lambda i: (0, i, 0)),
            pl.BlockSpec((BN, D), lambda i: (i, 0)),
            pl.BlockSpec((D, D), lambda i: (0, 0)),
            pl.BlockSpec((1, D), lambda i: (0, 0)),
            pl.BlockSpec((D, D), lambda i: (0, 0)),
        ],
        out_specs=pl.BlockSpec((BN, D), lambda i: (i, 0)),
        out_shape=jax.ShapeDtypeStruct((N, D), jnp.float32),
    )(psum, pcnt, x, W_l, b_l, W_r)


def kernel(x, edge_index, W_l, b_l, W_r):
    eidx = edge_index.reshape(2, NW, NCHUNKS, CHUNK)
    psum, pcnt = _sc_agg(x, eidx)
    return _tc_dense(psum, pcnt, x, W_l, b_l.reshape(1, D), W_r)


# speedup vs baseline: 1.0541x; 1.0123x over previous
import jax
import jax.numpy as jnp
from jax import lax
from jax.experimental import pallas as pl
from jax.experimental.pallas import tpu as pltpu
from jax.experimental.pallas import tpu_sc as plsc

N = 10000
E = 320000
D = 128

NC = 2
NS = 16
NW = NC * NS
EPW = E // NW
CHUNK = 125
NCHUNKS = EPW // CHUNK
WB = 80
NWB = N // WB
NH = 2
HC = NCHUNKS // NH


def _fill(buf, value):
    @pl.loop(0, CHUNK)
    def _(i):
        for j in range(D // 16):
            buf[i, pl.ds(j * 16, 16)] = jnp.full((16,), value, jnp.float32)


def _writeback(acc_sh, out_hbm, c, s, rows_a, rows_b, sg_a, sg_b):
    wa = rows_a.at[pl.ds(0, WB)]
    wb = rows_b.at[pl.ds(0, WB)]

    @pl.loop(s, NWB, step=2 * NS)
    def _(t):
        sl = pl.ds(t * WB, WB)
        pltpu.sync_copy(acc_sh.at[sl], wa)
        pltpu.async_copy(wa, out_hbm.at[c, sl], sg_a)

        @pl.when(t + NS < NWB)
        def _():
            sl2 = pl.ds((t + NS) * WB, WB)
            pltpu.sync_copy(acc_sh.at[sl2], wb)
            pltpu.async_copy(wb, out_hbm.at[c, sl2], sg_b)
            pltpu.make_async_copy(wb, out_hbm.at[c, sl2], sg_b).wait()

        pltpu.make_async_copy(wa, out_hbm.at[c, sl], sg_a).wait()


def _agg_body(x_hbm, eidx_hbm, psum_hbm, pcnt_hbm,
              src_i, dst_i, rows_a, rows_b, acc_sh, sg_a, sg_b):
    c = lax.axis_index("c")
    s = lax.axis_index("s")
    wid = c * NS + s

    _fill(rows_b, 0.0)

    @pl.loop(s, NWB, step=NS)
    def _(t):
        pltpu.sync_copy(rows_b.at[pl.ds(0, WB)],
                        acc_sh.at[pl.ds(t * WB, WB)])

    plsc.subcore_barrier()

    pltpu.sync_copy(eidx_hbm.at[1, wid], dst_i)

    def gather(j, buf, sem):
        pltpu.async_copy(x_hbm.at[src_i.at[j]], buf, sem)

    def gwait(j, buf, sem):
        pltpu.make_async_copy(x_hbm.at[src_i.at[j]], buf, sem).wait()

    bufs = [(rows_a, sg_a), (rows_b, sg_b)]
    NB = len(bufs)

    for h in range(NH):
        d0 = h * HC
        pltpu.sync_copy(eidx_hbm.at[0, wid, pl.ds(d0, HC)], src_i)
        for k in range(NB):
            gather(k, *bufs[k])

        @pl.loop(0, HC // NB - 1)
        def _(jg):
            j = jg * NB
            for k in range(NB):
                buf, sem = bufs[k]
                gwait(j + k, buf, sem)
                pltpu.sync_copy(buf, acc_sh.at[dst_i.at[d0 + j + k]],
                                add=True)
                gather(j + NB + k, buf, sem)

        jl = HC - NB
        for k in range(NB):
            buf, sem = bufs[k]
            gwait(jl + k, buf, sem)
            pltpu.sync_copy(buf, acc_sh.at[dst_i.at[d0 + jl + k]],
                            add=True)

    plsc.subcore_barrier()

    _writeback(acc_sh, psum_hbm, c, s, rows_a, rows_b, sg_a, sg_b)

    plsc.subcore_barrier()

    _fill(rows_a, 1.0)

    @pl.loop(0, NCHUNKS)
    def _(j):
        pltpu.sync_copy(rows_a, acc_sh.at[dst_i.at[j]], add=True)

    plsc.subcore_barrier()

    _writeback(acc_sh, pcnt_hbm, c, s, rows_a, rows_b, sg_a, sg_b)


_sc_agg = pl.kernel(
    _agg_body,
    out_type=[
        jax.ShapeDtypeStruct((NC, N, D), jnp.float32),
        jax.ShapeDtypeStruct((NC, N, D), jnp.float32),
    ],
    mesh=plsc.VectorSubcoreMesh(core_axis_name="c", subcore_axis_name="s"),
    scratch_types=[
        pltpu.VMEM((HC, CHUNK), jnp.int32),
        pltpu.VMEM((NCHUNKS, CHUNK), jnp.int32),
        pltpu.VMEM((CHUNK, D), jnp.float32),
        pltpu.VMEM((CHUNK, D), jnp.float32),
        pltpu.VMEM_SHARED((N, D), jnp.float32),
        pltpu.SemaphoreType.DMA,
        pltpu.SemaphoreType.DMA,
    ],
)


BN = 2000


def _tc_body(psum_ref, pcnt_ref, x_ref, wl_ref, bl_ref, wr_ref, out_ref):
    ssum = psum_ref[0] + psum_ref[1]
    cnt = (pcnt_ref[0][:, 0:1] + pcnt_ref[1][:, 0:1]) - ssum[:, 0:1]
    mean = ssum / jnp.maximum(cnt, 1.0)
    out = (jnp.dot(mean, wl_ref[:], preferred_element_type=jnp.float32)
           + bl_ref[:]
           + jnp.dot(x_ref[:], wr_ref[:], preferred_element_type=jnp.float32))
    nrm = jnp.sqrt(jnp.sum(out * out, axis=1, keepdims=True))
    out_ref[:] = out / jnp.maximum(nrm, 1e-12)


def _tc_dense(psum, pcnt, x, W_l, b_l, W_r):
    return pl.pallas_call(
        _tc_body,
        grid=(N // BN,),
        in_specs=[
            pl.BlockSpec((NC, BN, D), lambda i: (0, i, 0)),
            pl.BlockSpec((NC, BN, D), lambda i: (0, i, 0)),
            pl.BlockSpec((BN, D), lambda i: (i, 0)),
            pl.BlockSpec((D, D), lambda i: (0, 0)),
            pl.BlockSpec((1, D), lambda i: (0, 0)),
            pl.BlockSpec((D, D), lambda i: (0, 0)),
        ],
        out_specs=pl.BlockSpec((BN, D), lambda i: (i, 0)),
        out_shape=jax.ShapeDtypeStruct((N, D), jnp.float32),
    )(psum, pcnt, x, W_l, b_l, W_r)


def kernel(x, edge_index, W_l, b_l, W_r):
    eidx = edge_index.reshape(2, NW, NCHUNKS, CHUNK)
    psum, pcnt = _sc_agg(x, eidx)
    return _tc_dense(psum, pcnt, x, W_l, b_l.reshape(1, D), W_r)


# speedup vs baseline: 1.0569x; 1.0027x over previous
import jax
import jax.numpy as jnp
from jax import lax
from jax.experimental import pallas as pl
from jax.experimental.pallas import tpu as pltpu
from jax.experimental.pallas import tpu_sc as plsc

N = 10000
E = 320000
D = 128

NC = 2
NS = 16
NW = NC * NS
EPW = E // NW
CHUNK = 125
NCHUNKS = EPW // CHUNK
WB = 80
NWB = N // WB
NH = 2
HC = NCHUNKS // NH


def _fill(buf, value):
    @pl.loop(0, CHUNK)
    def _(i):
        for j in range(D // 16):
            buf[i, pl.ds(j * 16, 16)] = jnp.full((16,), value, jnp.float32)


def _writeback(acc_sh, out_hbm, c, s, rows_a, rows_b, sg_a, sg_b):
    wa = rows_a.at[pl.ds(0, WB)]
    wb = rows_b.at[pl.ds(0, WB)]

    @pl.loop(s, NWB, step=2 * NS)
    def _(t):
        sl = pl.ds(t * WB, WB)
        pltpu.sync_copy(acc_sh.at[sl], wa)
        pltpu.async_copy(wa, out_hbm.at[c, sl], sg_a)

        @pl.when(t + NS < NWB)
        def _():
            sl2 = pl.ds((t + NS) * WB, WB)
            pltpu.sync_copy(acc_sh.at[sl2], wb)
            pltpu.async_copy(wb, out_hbm.at[c, sl2], sg_b)
            pltpu.make_async_copy(wb, out_hbm.at[c, sl2], sg_b).wait()

        pltpu.make_async_copy(wa, out_hbm.at[c, sl], sg_a).wait()


def _agg_body(x_hbm, eidx_hbm, psum_hbm, pcnt_hbm,
              src_i, dst_i, rows_a, rows_b, acc_sh, sg_a, sg_b):
    c = lax.axis_index("c")
    s = lax.axis_index("s")
    wid = c * NS + s

    _fill(rows_b, 0.0)

    @pl.loop(s, NWB, step=NS)
    def _(t):
        pltpu.sync_copy(rows_b.at[pl.ds(0, WB)],
                        acc_sh.at[pl.ds(t * WB, WB)])

    plsc.subcore_barrier()

    pltpu.sync_copy(eidx_hbm.at[1, wid], dst_i)

    def gather(j, buf, sem):
        pltpu.async_copy(x_hbm.at[src_i.at[j]], buf, sem)

    def gwait(j, buf, sem):
        pltpu.make_async_copy(x_hbm.at[src_i.at[j]], buf, sem).wait()

    bufs = [(rows_a, sg_a), (rows_b, sg_b)]
    NB = len(bufs)

    for h in range(NH):
        d0 = h * HC
        pltpu.sync_copy(eidx_hbm.at[0, wid, pl.ds(d0, HC)], src_i)
        for k in range(NB):
            gather(k, *bufs[k])

        @pl.loop(0, HC // NB - 1)
        def _(jg):
            j = jg * NB
            for k in range(NB):
                buf, sem = bufs[k]
                gwait(j + k, buf, sem)
                pltpu.sync_copy(buf, acc_sh.at[dst_i.at[d0 + j + k]],
                                add=True)
                gather(j + NB + k, buf, sem)

        jl = HC - NB
        for k in range(NB):
            buf, sem = bufs[k]
            gwait(jl + k, buf, sem)
            pltpu.sync_copy(buf, acc_sh.at[dst_i.at[d0 + jl + k]],
                            add=True)

    plsc.subcore_barrier()

    _writeback(acc_sh, psum_hbm, c, s, rows_a, rows_b, sg_a, sg_b)

    plsc.subcore_barrier()

    _fill(rows_a, 1.0)

    @pl.loop(0, NCHUNKS)
    def _(j):
        pltpu.sync_copy(rows_a, acc_sh.at[dst_i.at[j]], add=True)

    plsc.subcore_barrier()

    _writeback(acc_sh, pcnt_hbm, c, s, rows_a, rows_b, sg_a, sg_b)


_sc_agg = pl.kernel(
    _agg_body,
    out_type=[
        jax.ShapeDtypeStruct((NC, N, D), jnp.float32),
        jax.ShapeDtypeStruct((NC, N, D), jnp.float32),
    ],
    mesh=plsc.VectorSubcoreMesh(core_axis_name="c", subcore_axis_name="s"),
    scratch_types=[
        pltpu.VMEM((HC, CHUNK), jnp.int32),
        pltpu.VMEM((NCHUNKS, CHUNK), jnp.int32),
        pltpu.VMEM((CHUNK, D), jnp.float32),
        pltpu.VMEM((CHUNK, D), jnp.float32),
        pltpu.VMEM_SHARED((N, D), jnp.float32),
        pltpu.SemaphoreType.DMA,
        pltpu.SemaphoreType.DMA,
    ],
)


BN = 5000


def _tc_body(psum_ref, pcnt_ref, x_ref, wl_ref, bl_ref, wr_ref, out_ref):
    ssum = psum_ref[0] + psum_ref[1]
    cnt = (pcnt_ref[0][:, 0:1] + pcnt_ref[1][:, 0:1]) - ssum[:, 0:1]
    mean = ssum / jnp.maximum(cnt, 1.0)
    out = (jnp.dot(mean, wl_ref[:], preferred_element_type=jnp.float32)
           + bl_ref[:]
           + jnp.dot(x_ref[:], wr_ref[:], preferred_element_type=jnp.float32))
    nrm = jnp.sqrt(jnp.sum(out * out, axis=1, keepdims=True))
    out_ref[:] = out / jnp.maximum(nrm, 1e-12)


def _tc_dense(psum, pcnt, x, W_l, b_l, W_r):
    return pl.pallas_call(
        _tc_body,
        grid=(N // BN,),
        in_specs=[
            pl.BlockSpec((NC, BN, D), lambda i: (0, i, 0)),
            pl.BlockSpec((NC, BN, D), lambda i: (0, i, 0)),
            pl.BlockSpec((BN, D), lambda i: (i, 0)),
            pl.BlockSpec((D, D), lambda i: (0, 0)),
            pl.BlockSpec((1, D), lambda i: (0, 0)),
            pl.BlockSpec((D, D), lambda i: (0, 0)),
        ],
        out_specs=pl.BlockSpec((BN, D), lambda i: (i, 0)),
        out_shape=jax.ShapeDtypeStruct((N, D), jnp.float32),
    )(psum, pcnt, x, W_l, b_l, W_r)


def kernel(x, edge_index, W_l, b_l, W_r):
    eidx = edge_index.reshape(2, NW, NCHUNKS, CHUNK)
    psum, pcnt = _sc_agg(x, eidx)
    return _tc_dense(psum, pcnt, x, W_l, b_l.reshape(1, D), W_r)
